# baseline (speedup 1.0000x reference)
import functools

import jax
import jax.numpy as jnp
from jax.experimental import pallas as pl
from jax.experimental.pallas import tpu as pltpu

_TEMPERATURE = 1.5
_C_TILE = 512


def _retrieve_body(x_ref, s_ref, mu_ref, std_ref, o_ref, *, min_std):
    x = x_ref[...].astype(jnp.bfloat16)
    s = s_ref[...].astype(jnp.bfloat16)
    scores = jax.lax.dot_general(
        x, s, (((1,), (1,)), ((), ())),
        preferred_element_type=jnp.float32)
    mu = mu_ref[...]
    std_safe = jnp.maximum(std_ref[...], min_std)
    z = (scores - mu) / std_safe
    z = jnp.where(jnp.isnan(z), 0.0, z)
    z = jnp.where(z == jnp.inf, 10.0, z)
    z = jnp.where(z == -jnp.inf, -10.0, z)
    o_ref[...] = z / _TEMPERATURE


def kernel(x_bits, S, z_mu, z_std):
    B, N = x_bits.shape
    C = S.shape[0]
    min_std = max(1e-06, 1.0 / (B ** 0.5)) if B > 0 else 1e-06
    mu2 = z_mu.reshape(1, C)
    std2 = z_std.reshape(1, C)
    grid = (pl.cdiv(C, _C_TILE),)
    return pl.pallas_call(
        functools.partial(_retrieve_body, min_std=min_std),
        grid=grid,
        in_specs=[
            pl.BlockSpec((B, N), lambda i: (0, 0)),
            pl.BlockSpec((_C_TILE, N), lambda i: (i, 0)),
            pl.BlockSpec((1, _C_TILE), lambda i: (0, i)),
            pl.BlockSpec((1, _C_TILE), lambda i: (0, i)),
        ],
        out_specs=pl.BlockSpec((B, _C_TILE), lambda i: (0, i)),
        out_shape=jax.ShapeDtypeStruct((B, C), jnp.float32),
        compiler_params=pltpu.CompilerParams(
            dimension_semantics=("arbitrary",),
        ),
    )(x_bits, S, mu2, std2)
